# baseline (speedup 1.0000x reference)
import jax
import jax.numpy as jnp
from jax.experimental import pallas as pl
from jax.experimental.pallas import tpu as pltpu


N_TOK = 8192
D_MODEL = 4096
N_GATES = 64
R = 512
NCHUNK = N_TOK // R
NQ = 4


def _router_kernel(x_hbm, wt_ref, probs_ref, scores_ref, idx_ref, *scratch):
    bufs = scratch[:NQ]
    sems = scratch[NQ:]

    def start_copy(c):
        q = c % NQ
        pltpu.make_async_copy(
            x_hbm.at[pl.ds(c * R, R), :], bufs[q], sems[q]
        ).start(priority=q % 2)

    def wait_copy(c):
        q = c % NQ
        pltpu.make_async_copy(
            x_hbm.at[pl.ds(c * R, R), :], bufs[q], sems[q]
        ).wait()

    for c in range(min(NQ, NCHUNK)):
        start_copy(c)

    wt = wt_ref[...]
    for c in range(NCHUNK):
        q = c % NQ
        wait_copy(c)
        logits = jnp.dot(bufs[q][...], wt, preferred_element_type=jnp.float32)
        m = jnp.max(logits, axis=-1, keepdims=True)
        e = jnp.exp(logits - m)
        s = jnp.sum(e, axis=-1, keepdims=True)
        probs = e / s
        probs_ref[pl.ds(c * R, R), :] = probs
        scores_ref[c, :] = jnp.max(probs, axis=-1)
        idx_ref[c, :] = jnp.argmax(probs, axis=-1).astype(jnp.int32)
        if c + NQ < NCHUNK:
            start_copy(c + NQ)


@jax.jit
def kernel(x, W):
    wt = W.T
    probs, scores, idx = pl.pallas_call(
        _router_kernel,
        grid=(),
        in_specs=[
            pl.BlockSpec(memory_space=pltpu.MemorySpace.HBM),
            pl.BlockSpec(memory_space=pltpu.MemorySpace.VMEM),
        ],
        out_specs=[
            pl.BlockSpec(memory_space=pltpu.MemorySpace.VMEM),
            pl.BlockSpec(memory_space=pltpu.MemorySpace.VMEM),
            pl.BlockSpec(memory_space=pltpu.MemorySpace.VMEM),
        ],
        out_shape=[
            jax.ShapeDtypeStruct((N_TOK, N_GATES), jnp.float32),
            jax.ShapeDtypeStruct((NCHUNK, R), jnp.float32),
            jax.ShapeDtypeStruct((NCHUNK, R), jnp.int32),
        ],
        scratch_shapes=[pltpu.VMEM((R, D_MODEL), jnp.float32) for _ in range(NQ)]
        + [pltpu.SemaphoreType.DMA for _ in range(NQ)],
    )(x, wt)
    return idx.reshape(N_TOK), scores.reshape(N_TOK), probs


# speedup vs baseline: 1.1564x; 1.1564x over previous
import jax
import jax.numpy as jnp
from jax.experimental import pallas as pl
from jax.experimental.pallas import tpu as pltpu


N_TOK = 8192
D_MODEL = 4096
N_GATES = 64
HALF = N_TOK // 2
R = 512
NSTEP = HALF // R
NB = 2


def _softmax_argmax(logits):
    m = jnp.max(logits, axis=-1, keepdims=True)
    e = jnp.exp(logits - m)
    s = jnp.sum(e, axis=-1, keepdims=True)
    probs = e / s
    scores = jnp.max(probs, axis=-1)
    idx = jnp.argmax(probs, axis=-1).astype(jnp.int32)
    return probs, scores, idx


def _router_kernel(
    x_top_ref,
    wt_ref,
    x_hbm,
    probs_t_ref,
    scores_t_ref,
    idx_t_ref,
    probs_b_ref,
    scores_b_ref,
    idx_b_ref,
    buf0,
    buf1,
    sem0,
    sem1,
):
    i = pl.program_id(0)
    bufs = (buf0, buf1)
    sems = (sem0, sem1)

    def copy(c, slot):
        return pltpu.make_async_copy(
            x_hbm.at[pl.ds(HALF + c * R, R), :], bufs[slot], sems[slot]
        )

    @pl.when(i == 0)
    def _prologue():
        copy(0, 0).start()
        copy(1, 1).start()

    wt = wt_ref[...]

    logits_t = jnp.dot(x_top_ref[...], wt, preferred_element_type=jnp.float32)
    probs, scores, idx = _softmax_argmax(logits_t)
    probs_t_ref[...] = probs
    scores_t_ref[0, 0, :] = scores
    idx_t_ref[0, 0, :] = idx

    for slot in range(NB):
        @pl.when(i % NB == slot)
        def _bottom():
            copy(i, slot).wait()
            logits_b = jnp.dot(bufs[slot][...], wt, preferred_element_type=jnp.float32)
            probs, scores, idx = _softmax_argmax(logits_b)
            probs_b_ref[pl.ds(i * R, R), :] = probs
            scores_b_ref[i, :] = scores
            idx_b_ref[i, :] = idx

            @pl.when(i + NB < NSTEP)
            def _refill():
                copy(i + NB, slot).start()


@jax.jit
def kernel(x, W):
    wt = W.T
    grid = (NSTEP,)
    pt, st, it_, pb, sb, ib = pl.pallas_call(
        _router_kernel,
        grid=grid,
        in_specs=[
            pl.BlockSpec((R, D_MODEL), lambda i: (i, 0)),
            pl.BlockSpec((D_MODEL, N_GATES), lambda i: (0, 0)),
            pl.BlockSpec(memory_space=pltpu.MemorySpace.HBM),
        ],
        out_specs=[
            pl.BlockSpec((R, N_GATES), lambda i: (i, 0)),
            pl.BlockSpec((1, 1, R), lambda i: (i, 0, 0)),
            pl.BlockSpec((1, 1, R), lambda i: (i, 0, 0)),
            pl.BlockSpec(memory_space=pltpu.MemorySpace.VMEM),
            pl.BlockSpec(memory_space=pltpu.MemorySpace.VMEM),
            pl.BlockSpec(memory_space=pltpu.MemorySpace.VMEM),
        ],
        out_shape=[
            jax.ShapeDtypeStruct((HALF, N_GATES), jnp.float32),
            jax.ShapeDtypeStruct((NSTEP, 1, R), jnp.float32),
            jax.ShapeDtypeStruct((NSTEP, 1, R), jnp.int32),
            jax.ShapeDtypeStruct((HALF, N_GATES), jnp.float32),
            jax.ShapeDtypeStruct((NSTEP, R), jnp.float32),
            jax.ShapeDtypeStruct((NSTEP, R), jnp.int32),
        ],
        scratch_shapes=[
            pltpu.VMEM((R, D_MODEL), jnp.float32),
            pltpu.VMEM((R, D_MODEL), jnp.float32),
            pltpu.SemaphoreType.DMA,
            pltpu.SemaphoreType.DMA,
        ],
    )(x, wt, x)
    idx = jnp.concatenate([it_.reshape(HALF), ib.reshape(HALF)])
    scores = jnp.concatenate([st.reshape(HALF), sb.reshape(HALF)])
    probs = jnp.concatenate([pt, pb], axis=0)
    return idx, scores, probs
